# baseline (speedup 1.0000x reference)
import functools
import math

import jax
import jax.numpy as jnp
from jax import lax
from jax.experimental import pallas as pl
from jax.experimental.pallas import tpu as pltpu
from jax.experimental.pallas import tpu_sc as plsc

E = 8
K = 2
D = 768
H = 12
DH = D // H
S = 2048
F = 4 * D

RB = 128
NPOS = 5120
NB = NPOS // RB
NBPAD = 64


def _router_body(lg_ref, pos1_ref, pos2_ref, w1_ref, w2_ref, eb_ref,
                 meta_ref, aux_ref):
    logits = lg_ref[...]
    m = jnp.max(logits, axis=1, keepdims=True)
    ex = jnp.exp(logits - m)
    probs = ex / jnp.sum(ex, axis=1, keepdims=True)

    lane = jax.lax.broadcasted_iota(jnp.int32, (S, E), 1)
    i1 = jnp.min(jnp.where(logits == m, lane, E), axis=1, keepdims=True)
    sel1 = lane == i1
    logits2 = jnp.where(sel1, jnp.float32(-jnp.inf), logits)
    m2 = jnp.max(logits2, axis=1, keepdims=True)
    i2 = jnp.min(jnp.where(logits2 == m2, lane, E), axis=1, keepdims=True)
    sel2 = lane == i2
    v1 = jnp.max(probs, axis=1, keepdims=True)
    v2 = jnp.max(jnp.where(sel1, -1.0, probs), axis=1, keepdims=True)
    wsum = jnp.maximum(v1 + v2, 1e-9)
    w1_ref[...] = v1 / wsum
    w2_ref[...] = v2 / wsum

    A = (sel1 | sel2).astype(jnp.float32)
    c = A
    sh = 1
    while sh < S:
        c = c + jnp.concatenate(
            [jnp.zeros((sh, E), jnp.float32), c[:S - sh]], axis=0)
        sh *= 2
    rank = c - A
    n = c[S - 1:S, :]
    mpad = jnp.floor((n + (RB - 1.0)) * (1.0 / RB)) * float(RB)
    mc = mpad
    sh = 1
    while sh < E:
        mc = mc + jnp.concatenate(
            [jnp.zeros((1, sh), jnp.float32), mc[:, :E - sh]], axis=1)
        sh *= 2
    off = mc - mpad
    total = mc[:, E - 1:E]

    pos = off + rank
    p1 = jnp.sum(jnp.where(sel1, pos, 0.0), axis=1, keepdims=True)
    p2 = jnp.sum(jnp.where(sel2, pos, 0.0), axis=1, keepdims=True)
    pos1_ref[...] = p1.astype(jnp.int32)
    pos2_ref[...] = p2.astype(jnp.int32)

    bi = jax.lax.broadcasted_iota(jnp.int32, (1, NBPAD), 1).astype(jnp.float32) * float(RB)
    end = off + mpad
    acc = jnp.zeros((1, NBPAD), jnp.float32)
    for e in range(E):
        acc = acc + (bi >= end[:, e:e + 1]).astype(jnp.float32)
    eb_ref[...] = jnp.minimum(acc, float(E - 1)).astype(jnp.int32)

    r0 = jnp.concatenate([off, total, jnp.zeros((1, 7), jnp.float32)], axis=1)
    r1 = jnp.concatenate([n, jnp.zeros((1, 8), jnp.float32)], axis=1)
    meta_ref[...] = jnp.concatenate([r0, r1], axis=0).astype(jnp.int32)

    f = jnp.sum(sel1.astype(jnp.float32), axis=0, keepdims=True) / S
    p_mean = jnp.sum(probs, axis=0, keepdims=True) / S
    aux_ref[...] = float(E) * jnp.sum(f * p_mean, axis=1, keepdims=True)


def _ln(x, g, b, eps=1e-5):
    mu = jnp.mean(x, axis=-1, keepdims=True)
    xc = x - mu
    var = jnp.mean(xc * xc, axis=-1, keepdims=True)
    return xc * jax.lax.rsqrt(var + eps) * g + b


def _gexpert_body(eb_ref, xg_ref, xkv_ref, w_ref, wqkv_ref, bqkv_ref,
                  wo_ref, bo_ref, g1_ref, b1_ref, wf1_ref, bf1_ref, wf2_ref,
                  bf2_ref, g2_ref, b2_ref, g3_ref, b3_ref, out_ref,
                  k_scr, v_scr):
    b = pl.program_id(0)
    cdims = (((1,), (1,)), ((), ()))
    wqkv = wqkv_ref[0]
    brow = bqkv_ref[0, 0, :][None, :]

    e_prev = eb_ref[jnp.maximum(b - 1, 0)]
    recompute = jnp.logical_or(b == 0, eb_ref[b] != e_prev)

    @pl.when(recompute)
    def _():
        xkv = xkv_ref[...]
        k = jax.lax.dot_general(xkv, wqkv[D:2 * D], cdims,
                                preferred_element_type=jnp.float32)
        v = jax.lax.dot_general(xkv, wqkv[2 * D:], cdims,
                                preferred_element_type=jnp.float32)
        k_scr[...] = (k + brow[:, D:2 * D]).astype(jnp.bfloat16)
        v_scr[...] = (v + brow[:, 2 * D:]).astype(jnp.bfloat16)

    xq = xg_ref[...]
    q = jax.lax.dot_general(xq, wqkv[:D], cdims,
                            preferred_element_type=jnp.float32)
    qb = (q + brow[:, :D]).astype(jnp.bfloat16)

    heads = []
    for h in range(H):
        cols = slice(h * DH, (h + 1) * DH)
        s = jax.lax.dot_general(qb[:, cols], k_scr[:, cols], cdims,
                                preferred_element_type=jnp.float32)
        s = s * (1.0 / math.sqrt(DH))
        mx = jnp.max(s, axis=1, keepdims=True)
        p = jnp.exp(s - mx)
        ps = jnp.sum(p, axis=1, keepdims=True)
        o = jax.lax.dot_general(p.astype(jnp.bfloat16), v_scr[:, cols],
                                (((1,), (0,)), ((), ())),
                                preferred_element_type=jnp.float32)
        heads.append((o / ps).astype(jnp.bfloat16))
    attn = jnp.concatenate(heads, axis=1)

    xb = xq.astype(jnp.float32)
    a = jax.lax.dot_general(attn, wo_ref[0], cdims,
                            preferred_element_type=jnp.float32)
    a = a + bo_ref[0, 0, :][None, :]
    h1 = _ln(xb + a, g1_ref[0, 0, :][None, :], b1_ref[0, 0, :][None, :])
    ff = jax.lax.dot_general(h1.astype(jnp.bfloat16), wf1_ref[0], cdims,
                             preferred_element_type=jnp.float32)
    ff = ff + bf1_ref[0, 0, :][None, :]
    ff = 0.5 * ff * (1.0 + jax.lax.erf(ff * (1.0 / math.sqrt(2.0))))
    y = jax.lax.dot_general(ff.astype(jnp.bfloat16), wf2_ref[0], cdims,
                            preferred_element_type=jnp.float32)
    y = y + bf2_ref[0, 0, :][None, :]
    t = _ln(h1 + y, g2_ref[0, 0, :][None, :], b2_ref[0, 0, :][None, :])
    z = _ln(t + xb, g3_ref[0, 0, :][None, :], b3_ref[0, 0, :][None, :])
    out_ref[...] = z * w_ref[...]


def _rep8(b):
    return jnp.broadcast_to(b[:, None, :], (b.shape[0], 8, b.shape[1]))


NC = 2
NS = 16
NW = NC * NS
TOKT = S // NW
POST = NPOS // NW
NTRASH = 8
_SC_MESH = plsc.VectorSubcoreMesh(core_axis_name="c", subcore_axis_name="s")


def _wid():
    return lax.axis_index("s") * NC + lax.axis_index("c")


@functools.partial(
    pl.kernel, mesh=_SC_MESH,
    compiler_params=pltpu.CompilerParams(needs_layout_passes=False),
    out_type=[
        jax.ShapeDtypeStruct((NPOS + NTRASH,), jnp.int32),
        jax.ShapeDtypeStruct((NPOS + NTRASH,), jnp.float32),
    ],
    scratch_types=[
        pltpu.VMEM((TOKT,), jnp.int32),
        pltpu.VMEM((TOKT,), jnp.int32),
        pltpu.VMEM((TOKT,), jnp.int32),
        pltpu.VMEM((TOKT,), jnp.float32),
        pltpu.VMEM((TOKT,), jnp.float32),
        pltpu.VMEM((RB,), jnp.int32),
        pltpu.VMEM((RB,), jnp.int32),
        pltpu.VMEM((RB,), jnp.float32),
        pltpu.VMEM((16,), jnp.int32),
        pltpu.VMEM((16,), jnp.int32),
    ],
)
def _sc_scatter(pos1_hbm, pos2_hbm, w1_hbm, w2_hbm, meta_hbm,
                tokpos_hbm, wpos_hbm,
                idx1_v, idx2_v, tok_v, wv1_v, wv2_v, fidx_v, fsrc_v,
                fsrcw_v, m0_v, m1_v):
    wid = _wid()
    base = wid * TOKT
    pltpu.sync_copy(pos1_hbm.at[pl.ds(base, TOKT)], idx1_v)
    pltpu.sync_copy(pos2_hbm.at[pl.ds(base, TOKT)], idx2_v)
    pltpu.sync_copy(w1_hbm.at[pl.ds(base, TOKT)], wv1_v)
    pltpu.sync_copy(w2_hbm.at[pl.ds(base, TOKT)], wv2_v)
    for c in range(TOKT // 16):
        tok_v[pl.ds(c * 16, 16)] = (
            jnp.arange(16, dtype=jnp.int32) + (base + c * 16))
    pltpu.sync_copy(tok_v, tokpos_hbm.at[idx1_v])
    pltpu.sync_copy(tok_v, tokpos_hbm.at[idx2_v])
    pltpu.sync_copy(wv1_v, wpos_hbm.at[idx1_v])
    pltpu.sync_copy(wv2_v, wpos_hbm.at[idx2_v])

    trash = NPOS + lax.rem(wid, NTRASH)

    @pl.when(wid < E)
    def _():
        pltpu.sync_copy(meta_hbm.at[0], m0_v)
        pltpu.sync_copy(meta_hbm.at[1], m1_v)
        lanes = jnp.arange(16, dtype=jnp.int32)
        offv = jnp.max(jnp.where(lanes == wid, m0_v[...], 0), axis=0)
        nv = jnp.max(jnp.where(lanes == wid, m1_v[...], 0), axis=0)
        mv = ((nv + (RB - 1)) // RB) * RB
        for c in range(RB // 16):
            j = jnp.arange(16, dtype=jnp.int32) + c * 16
            valid = (nv + j) < mv
            fidx_v[pl.ds(c * 16, 16)] = jnp.where(
                valid, offv + nv + j, trash)
            fsrc_v[pl.ds(c * 16, 16)] = jnp.zeros((16,), jnp.int32)
            fsrcw_v[pl.ds(c * 16, 16)] = jnp.zeros((16,), jnp.float32)
        pltpu.sync_copy(fsrc_v, tokpos_hbm.at[fidx_v])
        pltpu.sync_copy(fsrcw_v, wpos_hbm.at[fidx_v])

    @pl.when(jnp.logical_and(wid >= E, wid < 2 * E))
    def _():
        pltpu.sync_copy(meta_hbm.at[0], m0_v)
        lanes = jnp.arange(16, dtype=jnp.int32)
        totv = jnp.max(jnp.where(lanes == E, m0_v[...], 0), axis=0)
        for c in range(RB // 16):
            p = (jnp.arange(16, dtype=jnp.int32)
                 + (2 * S + (wid - E) * RB + c * 16))
            fidx_v[pl.ds(c * 16, 16)] = jnp.where(p >= totv, p, trash)
            fsrc_v[pl.ds(c * 16, 16)] = jnp.zeros((16,), jnp.int32)
            fsrcw_v[pl.ds(c * 16, 16)] = jnp.zeros((16,), jnp.float32)
        pltpu.sync_copy(fsrc_v, tokpos_hbm.at[fidx_v])
        pltpu.sync_copy(fsrcw_v, wpos_hbm.at[fidx_v])


_GCH = 40
_NCH = POST // _GCH


@functools.partial(
    pl.kernel, mesh=_SC_MESH,
    compiler_params=pltpu.CompilerParams(needs_layout_passes=False),
    out_type=jax.ShapeDtypeStruct((NPOS, D // 2), jnp.int32),
    scratch_types=[
        pltpu.VMEM((POST,), jnp.int32),
        pltpu.VMEM((_NCH, _GCH, D // 2), jnp.int32),
        pltpu.SemaphoreType.DMA,
        pltpu.SemaphoreType.DMA,
    ],
)
def _sc_gather(tokpos_hbm, x_hbm, xg_hbm, cidx_v, rows_v, gsem, wsem):
    wid = _wid()
    base = wid * POST
    pltpu.sync_copy(tokpos_hbm.at[pl.ds(base, POST)], cidx_v)
    gets = [pltpu.async_copy(
        x_hbm.at[cidx_v.at[pl.ds(c * _GCH, _GCH)]], rows_v.at[c], gsem)
        for c in range(_NCH)]
    for g in gets:
        g.wait()
    puts = [pltpu.async_copy(
        rows_v.at[c], xg_hbm.at[pl.ds(base + c * _GCH, _GCH)], wsem)
        for c in range(_NCH)]
    for p in puts:
        p.wait()


@functools.partial(
    pl.kernel, mesh=_SC_MESH,
    compiler_params=pltpu.CompilerParams(needs_layout_passes=False),
    out_type=jax.ShapeDtypeStruct((S, D), jnp.float32),
    scratch_types=[
        pltpu.VMEM((TOKT,), jnp.int32),
        pltpu.VMEM((TOKT,), jnp.int32),
        pltpu.VMEM((TOKT, D), jnp.float32),
        pltpu.VMEM((TOKT, D), jnp.float32),
        pltpu.SemaphoreType.DMA,
    ],
)
def _sc_combine(zg_hbm, pos1_hbm, pos2_hbm, out_hbm,
                i1_v, i2_v, b1_v, b2_v, sem):
    wid = _wid()
    base = wid * TOKT
    pltpu.sync_copy(pos1_hbm.at[pl.ds(base, TOKT)], i1_v)
    pltpu.sync_copy(pos2_hbm.at[pl.ds(base, TOKT)], i2_v)
    g1 = pltpu.async_copy(zg_hbm.at[i1_v], b1_v, sem)
    g2 = pltpu.async_copy(zg_hbm.at[i2_v], b2_v, sem)
    g1.wait()
    g2.wait()

    def row_body(r, carry):
        for c in range(D // 16):
            sl = pl.ds(c * 16, 16)
            b1_v[r, sl] = b1_v[r, sl] + b2_v[r, sl]
        return carry

    lax.fori_loop(0, TOKT, row_body, 0)
    pltpu.sync_copy(b1_v, out_hbm.at[pl.ds(base, TOKT)])


def kernel(x, Wr, br, Wqkv, bqkv, Wo, bo, g1, b1, Wf1, bf1, Wf2, bf2,
           g2, b2, g3, b3):
    xs = x[0]
    xbf = xs.astype(jnp.bfloat16)

    logits = (x @ Wr.T + br)[0]

    pos1, pos2, w1c, w2c, eb, meta, aux = pl.pallas_call(
        _router_body,
        grid=(1,),
        in_specs=[pl.BlockSpec((S, E), lambda i: (0, 0))],
        out_specs=[
            pl.BlockSpec((S, 1), lambda i: (0, 0)),
            pl.BlockSpec((S, 1), lambda i: (0, 0)),
            pl.BlockSpec((S, 1), lambda i: (0, 0)),
            pl.BlockSpec((S, 1), lambda i: (0, 0)),
            pl.BlockSpec((1, NBPAD), lambda i: (0, 0)),
            pl.BlockSpec((2, 16), lambda i: (0, 0)),
            pl.BlockSpec((1, 1), lambda i: (0, 0)),
        ],
        out_shape=[
            jax.ShapeDtypeStruct((S, 1), jnp.int32),
            jax.ShapeDtypeStruct((S, 1), jnp.int32),
            jax.ShapeDtypeStruct((S, 1), jnp.float32),
            jax.ShapeDtypeStruct((S, 1), jnp.float32),
            jax.ShapeDtypeStruct((1, NBPAD), jnp.int32),
            jax.ShapeDtypeStruct((2, 16), jnp.int32),
            jax.ShapeDtypeStruct((1, 1), jnp.float32),
        ],
    )(logits)

    p1 = pos1.reshape(S)
    p2 = pos2.reshape(S)

    tokpos, wpos = _sc_scatter(p1, p2, w1c.reshape(S), w2c.reshape(S), meta)
    x_i32 = jax.lax.bitcast_convert_type(
        xbf.reshape(S, D // 2, 2), jnp.int32)
    xg_i32 = _sc_gather(tokpos, x_i32)
    xg = jax.lax.bitcast_convert_type(
        xg_i32, jnp.bfloat16).reshape(NPOS, D)
    wpos = wpos[:NPOS]

    zg = pl.pallas_call(
        _gexpert_body,
        grid_spec=pltpu.PrefetchScalarGridSpec(
            num_scalar_prefetch=1,
            grid=(NB,),
            in_specs=[
                pl.BlockSpec((RB, D), lambda b, eb: (b, 0)),
                pl.BlockSpec((S, D), lambda b, eb: (0, 0)),
                pl.BlockSpec((RB, 1), lambda b, eb: (b, 0)),
                pl.BlockSpec((1, 3 * D, D), lambda b, eb: (eb[b], 0, 0)),
                pl.BlockSpec((1, 8, 3 * D), lambda b, eb: (eb[b], 0, 0)),
                pl.BlockSpec((1, D, D), lambda b, eb: (eb[b], 0, 0)),
                pl.BlockSpec((1, 8, D), lambda b, eb: (eb[b], 0, 0)),
                pl.BlockSpec((1, 8, D), lambda b, eb: (eb[b], 0, 0)),
                pl.BlockSpec((1, 8, D), lambda b, eb: (eb[b], 0, 0)),
                pl.BlockSpec((1, F, D), lambda b, eb: (eb[b], 0, 0)),
                pl.BlockSpec((1, 8, F), lambda b, eb: (eb[b], 0, 0)),
                pl.BlockSpec((1, D, F), lambda b, eb: (eb[b], 0, 0)),
                pl.BlockSpec((1, 8, D), lambda b, eb: (eb[b], 0, 0)),
                pl.BlockSpec((1, 8, D), lambda b, eb: (eb[b], 0, 0)),
                pl.BlockSpec((1, 8, D), lambda b, eb: (eb[b], 0, 0)),
                pl.BlockSpec((1, 8, D), lambda b, eb: (eb[b], 0, 0)),
                pl.BlockSpec((1, 8, D), lambda b, eb: (eb[b], 0, 0)),
            ],
            out_specs=pl.BlockSpec((RB, D), lambda b, eb: (b, 0)),
            scratch_shapes=[
                pltpu.VMEM((S, D), jnp.bfloat16),
                pltpu.VMEM((S, D), jnp.bfloat16),
            ],
        ),
        out_shape=jax.ShapeDtypeStruct((NPOS, D), jnp.float32),
    )(eb[0], xg, xbf, wpos.reshape(NPOS, 1),
      Wqkv.astype(jnp.bfloat16), _rep8(bqkv),
      Wo.astype(jnp.bfloat16), _rep8(bo), _rep8(g1), _rep8(b1),
      Wf1.astype(jnp.bfloat16), _rep8(bf1), Wf2.astype(jnp.bfloat16),
      _rep8(bf2), _rep8(g2), _rep8(b2), _rep8(g3), _rep8(b3))

    out2d = _sc_combine(zg, p1, p2)

    return out2d[None], aux[0, 0]


# speedup vs baseline: 1.0234x; 1.0234x over previous
import functools
import math

import jax
import jax.numpy as jnp
from jax import lax
from jax.experimental import pallas as pl
from jax.experimental.pallas import tpu as pltpu
from jax.experimental.pallas import tpu_sc as plsc

E = 8
K = 2
D = 768
H = 12
DH = D // H
S = 2048
F = 4 * D

RB = 128
NPOS = 5120
NB = NPOS // RB
NBPAD = 64


def _router_body(lg_ref, pos1_ref, pos2_ref, w1_ref, w2_ref, eb_ref,
                 meta_ref, aux_ref):
    logits = lg_ref[...]
    m = jnp.max(logits, axis=1, keepdims=True)
    ex = jnp.exp(logits - m)
    probs = ex / jnp.sum(ex, axis=1, keepdims=True)

    lane = jax.lax.broadcasted_iota(jnp.int32, (S, E), 1)
    i1 = jnp.min(jnp.where(logits == m, lane, E), axis=1, keepdims=True)
    sel1 = lane == i1
    logits2 = jnp.where(sel1, jnp.float32(-jnp.inf), logits)
    m2 = jnp.max(logits2, axis=1, keepdims=True)
    i2 = jnp.min(jnp.where(logits2 == m2, lane, E), axis=1, keepdims=True)
    sel2 = lane == i2
    v1 = jnp.max(probs, axis=1, keepdims=True)
    v2 = jnp.max(jnp.where(sel1, -1.0, probs), axis=1, keepdims=True)
    wsum = jnp.maximum(v1 + v2, 1e-9)
    w1_ref[...] = v1 / wsum
    w2_ref[...] = v2 / wsum

    A = (sel1 | sel2).astype(jnp.float32)
    c = A
    sh = 1
    while sh < S:
        c = c + jnp.concatenate(
            [jnp.zeros((sh, E), jnp.float32), c[:S - sh]], axis=0)
        sh *= 2
    rank = c - A
    n = c[S - 1:S, :]
    mpad = jnp.floor((n + (RB - 1.0)) * (1.0 / RB)) * float(RB)
    mc = mpad
    sh = 1
    while sh < E:
        mc = mc + jnp.concatenate(
            [jnp.zeros((1, sh), jnp.float32), mc[:, :E - sh]], axis=1)
        sh *= 2
    off = mc - mpad
    total = mc[:, E - 1:E]

    pos = off + rank
    p1 = jnp.sum(jnp.where(sel1, pos, 0.0), axis=1, keepdims=True)
    p2 = jnp.sum(jnp.where(sel2, pos, 0.0), axis=1, keepdims=True)
    pos1_ref[...] = p1.astype(jnp.int32)
    pos2_ref[...] = p2.astype(jnp.int32)

    bi = jax.lax.broadcasted_iota(jnp.int32, (1, NBPAD), 1).astype(jnp.float32) * float(RB)
    end = off + mpad
    acc = jnp.zeros((1, NBPAD), jnp.float32)
    for e in range(E):
        acc = acc + (bi >= end[:, e:e + 1]).astype(jnp.float32)
    ebv = jnp.minimum(acc, float(E - 1))
    lanei = jax.lax.broadcasted_iota(jnp.int32, (1, NBPAD), 1)
    ebv = jnp.where(lanei == NBPAD - 1, total * (1.0 / RB), ebv)
    eb_ref[...] = ebv.astype(jnp.int32)

    r0 = jnp.concatenate([off, total, jnp.zeros((1, 7), jnp.float32)], axis=1)
    r1 = jnp.concatenate([n, jnp.zeros((1, 8), jnp.float32)], axis=1)
    meta_ref[...] = jnp.concatenate([r0, r1], axis=0).astype(jnp.int32)

    f = jnp.sum(sel1.astype(jnp.float32), axis=0, keepdims=True) / S
    p_mean = jnp.sum(probs, axis=0, keepdims=True) / S
    aux_ref[...] = float(E) * jnp.sum(f * p_mean, axis=1, keepdims=True)


def _ln(x, g, b, eps=1e-5):
    mu = jnp.mean(x, axis=-1, keepdims=True)
    xc = x - mu
    var = jnp.mean(xc * xc, axis=-1, keepdims=True)
    return xc * jax.lax.rsqrt(var + eps) * g + b


def _kv_body(xkv_ref, wkv_ref, bkv_ref, kv_ref):
    cdims = (((1,), (1,)), ((), ()))
    kv = jax.lax.dot_general(xkv_ref[...], wkv_ref[0], cdims,
                             preferred_element_type=jnp.float32)
    kv_ref[0] = (kv + bkv_ref[0, 0, :][None, :]).astype(jnp.bfloat16)


def _gexpert_body(eb_ref, xg_ref, kv_ref, w_ref, wqkv_ref, bqkv_ref,
                  wo_ref, bo_ref, g1_ref, b1_ref, wf1_ref, bf1_ref, wf2_ref,
                  bf2_ref, g2_ref, b2_ref, g3_ref, b3_ref, out_ref):
    b = pl.program_id(0)

    @pl.when(b < eb_ref[NBPAD - 1])
    def _():
        cdims = (((1,), (1,)), ((), ()))
        wqkv = wqkv_ref[0]
        brow = bqkv_ref[0, 0, :][None, :]

        xq = xg_ref[...]
        q = jax.lax.dot_general(xq, wqkv, cdims,
                                preferred_element_type=jnp.float32)
        qb = (q + brow[:, :D]).astype(jnp.bfloat16)

        heads = []
        for h in range(H):
            cols = slice(h * DH, (h + 1) * DH)
            colsv = slice(D + h * DH, D + (h + 1) * DH)
            s = jax.lax.dot_general(qb[:, cols], kv_ref[0][:, cols], cdims,
                                    preferred_element_type=jnp.float32)
            s = s * (1.0 / math.sqrt(DH))
            mx = jnp.max(s, axis=1, keepdims=True)
            p = jnp.exp(s - mx)
            ps = jnp.sum(p, axis=1, keepdims=True)
            o = jax.lax.dot_general(p.astype(jnp.bfloat16), kv_ref[0][:, colsv],
                                    (((1,), (0,)), ((), ())),
                                    preferred_element_type=jnp.float32)
            heads.append((o / ps).astype(jnp.bfloat16))
        attn = jnp.concatenate(heads, axis=1)

        xb = xq.astype(jnp.float32)
        a = jax.lax.dot_general(attn, wo_ref[0], cdims,
                                preferred_element_type=jnp.float32)
        a = a + bo_ref[0, 0, :][None, :]
        h1 = _ln(xb + a, g1_ref[0, 0, :][None, :], b1_ref[0, 0, :][None, :])
        ff = jax.lax.dot_general(h1.astype(jnp.bfloat16), wf1_ref[0], cdims,
                                 preferred_element_type=jnp.float32)
        ff = ff + bf1_ref[0, 0, :][None, :]
        ff = 0.5 * ff * (1.0 + jax.lax.erf(ff * (1.0 / math.sqrt(2.0))))
        y = jax.lax.dot_general(ff.astype(jnp.bfloat16), wf2_ref[0], cdims,
                                preferred_element_type=jnp.float32)
        y = y + bf2_ref[0, 0, :][None, :]
        t = _ln(h1 + y, g2_ref[0, 0, :][None, :], b2_ref[0, 0, :][None, :])
        z = _ln(t + xb, g3_ref[0, 0, :][None, :], b3_ref[0, 0, :][None, :])
        out_ref[...] = z * w_ref[...]


def _rep8(b):
    return jnp.broadcast_to(b[:, None, :], (b.shape[0], 8, b.shape[1]))


NC = 2
NS = 16
NW = NC * NS
TOKT = S // NW
POST = NPOS // NW
NTRASH = 8
_SC_MESH = plsc.VectorSubcoreMesh(core_axis_name="c", subcore_axis_name="s")


def _wid():
    return lax.axis_index("s") * NC + lax.axis_index("c")


@functools.partial(
    pl.kernel, mesh=_SC_MESH,
    compiler_params=pltpu.CompilerParams(needs_layout_passes=False),
    out_type=[
        jax.ShapeDtypeStruct((NPOS + NTRASH,), jnp.int32),
        jax.ShapeDtypeStruct((NPOS + NTRASH,), jnp.float32),
    ],
    scratch_types=[
        pltpu.VMEM((TOKT,), jnp.int32),
        pltpu.VMEM((TOKT,), jnp.int32),
        pltpu.VMEM((TOKT,), jnp.int32),
        pltpu.VMEM((TOKT,), jnp.float32),
        pltpu.VMEM((TOKT,), jnp.float32),
        pltpu.VMEM((RB,), jnp.int32),
        pltpu.VMEM((RB,), jnp.int32),
        pltpu.VMEM((RB,), jnp.float32),
        pltpu.VMEM((16,), jnp.int32),
        pltpu.VMEM((16,), jnp.int32),
    ],
)
def _sc_scatter(pos1_hbm, pos2_hbm, w1_hbm, w2_hbm, meta_hbm,
                tokpos_hbm, wpos_hbm,
                idx1_v, idx2_v, tok_v, wv1_v, wv2_v, fidx_v, fsrc_v,
                fsrcw_v, m0_v, m1_v):
    wid = _wid()
    base = wid * TOKT
    pltpu.sync_copy(pos1_hbm.at[pl.ds(base, TOKT)], idx1_v)
    pltpu.sync_copy(pos2_hbm.at[pl.ds(base, TOKT)], idx2_v)
    pltpu.sync_copy(w1_hbm.at[pl.ds(base, TOKT)], wv1_v)
    pltpu.sync_copy(w2_hbm.at[pl.ds(base, TOKT)], wv2_v)
    for c in range(TOKT // 16):
        tok_v[pl.ds(c * 16, 16)] = (
            jnp.arange(16, dtype=jnp.int32) + (base + c * 16))
    pltpu.sync_copy(tok_v, tokpos_hbm.at[idx1_v])
    pltpu.sync_copy(tok_v, tokpos_hbm.at[idx2_v])
    pltpu.sync_copy(wv1_v, wpos_hbm.at[idx1_v])
    pltpu.sync_copy(wv2_v, wpos_hbm.at[idx2_v])

    trash = NPOS + lax.rem(wid, NTRASH)

    @pl.when(wid < E)
    def _():
        pltpu.sync_copy(meta_hbm.at[0], m0_v)
        pltpu.sync_copy(meta_hbm.at[1], m1_v)
        lanes = jnp.arange(16, dtype=jnp.int32)
        offv = jnp.max(jnp.where(lanes == wid, m0_v[...], 0), axis=0)
        nv = jnp.max(jnp.where(lanes == wid, m1_v[...], 0), axis=0)
        mv = ((nv + (RB - 1)) // RB) * RB
        for c in range(RB // 16):
            j = jnp.arange(16, dtype=jnp.int32) + c * 16
            valid = (nv + j) < mv
            fidx_v[pl.ds(c * 16, 16)] = jnp.where(
                valid, offv + nv + j, trash)
            fsrc_v[pl.ds(c * 16, 16)] = jnp.zeros((16,), jnp.int32)
            fsrcw_v[pl.ds(c * 16, 16)] = jnp.zeros((16,), jnp.float32)
        pltpu.sync_copy(fsrc_v, tokpos_hbm.at[fidx_v])
        pltpu.sync_copy(fsrcw_v, wpos_hbm.at[fidx_v])

    @pl.when(jnp.logical_and(wid >= E, wid < 2 * E))
    def _():
        pltpu.sync_copy(meta_hbm.at[0], m0_v)
        lanes = jnp.arange(16, dtype=jnp.int32)
        totv = jnp.max(jnp.where(lanes == E, m0_v[...], 0), axis=0)
        for c in range(RB // 16):
            p = (jnp.arange(16, dtype=jnp.int32)
                 + (2 * S + (wid - E) * RB + c * 16))
            fidx_v[pl.ds(c * 16, 16)] = jnp.where(p >= totv, p, trash)
            fsrc_v[pl.ds(c * 16, 16)] = jnp.zeros((16,), jnp.int32)
            fsrcw_v[pl.ds(c * 16, 16)] = jnp.zeros((16,), jnp.float32)
        pltpu.sync_copy(fsrc_v, tokpos_hbm.at[fidx_v])
        pltpu.sync_copy(fsrcw_v, wpos_hbm.at[fidx_v])


_GCH = 40
_NCH = POST // _GCH


@functools.partial(
    pl.kernel, mesh=_SC_MESH,
    compiler_params=pltpu.CompilerParams(needs_layout_passes=False),
    out_type=jax.ShapeDtypeStruct((NPOS, D // 2), jnp.int32),
    scratch_types=[
        pltpu.VMEM((POST,), jnp.int32),
        pltpu.VMEM((_NCH, _GCH, D // 2), jnp.int32),
        pltpu.SemaphoreType.DMA,
        pltpu.SemaphoreType.DMA,
    ],
)
def _sc_gather(tokpos_hbm, x_hbm, xg_hbm, cidx_v, rows_v, gsem, wsem):
    wid = _wid()
    base = wid * POST
    pltpu.sync_copy(tokpos_hbm.at[pl.ds(base, POST)], cidx_v)
    gets = [pltpu.async_copy(
        x_hbm.at[cidx_v.at[pl.ds(c * _GCH, _GCH)]], rows_v.at[c], gsem)
        for c in range(_NCH)]
    for g in gets:
        g.wait()
    puts = [pltpu.async_copy(
        rows_v.at[c], xg_hbm.at[pl.ds(base + c * _GCH, _GCH)], wsem)
        for c in range(_NCH)]
    for p in puts:
        p.wait()


@functools.partial(
    pl.kernel, mesh=_SC_MESH,
    compiler_params=pltpu.CompilerParams(needs_layout_passes=False),
    out_type=jax.ShapeDtypeStruct((S, D), jnp.float32),
    scratch_types=[
        pltpu.VMEM((TOKT,), jnp.int32),
        pltpu.VMEM((TOKT,), jnp.int32),
        pltpu.VMEM((TOKT, D), jnp.float32),
        pltpu.VMEM((TOKT, D), jnp.float32),
        pltpu.SemaphoreType.DMA,
    ],
)
def _sc_combine(zg_hbm, pos1_hbm, pos2_hbm, out_hbm,
                i1_v, i2_v, b1_v, b2_v, sem):
    wid = _wid()
    base = wid * TOKT
    pltpu.sync_copy(pos1_hbm.at[pl.ds(base, TOKT)], i1_v)
    pltpu.sync_copy(pos2_hbm.at[pl.ds(base, TOKT)], i2_v)
    g1 = pltpu.async_copy(zg_hbm.at[i1_v], b1_v, sem)
    g2 = pltpu.async_copy(zg_hbm.at[i2_v], b2_v, sem)
    g1.wait()
    g2.wait()

    def row_body(r, carry):
        for c in range(D // 16):
            sl = pl.ds(c * 16, 16)
            b1_v[r, sl] = b1_v[r, sl] + b2_v[r, sl]
        return carry

    lax.fori_loop(0, TOKT, row_body, 0)
    pltpu.sync_copy(b1_v, out_hbm.at[pl.ds(base, TOKT)])


def kernel(x, Wr, br, Wqkv, bqkv, Wo, bo, g1, b1, Wf1, bf1, Wf2, bf2,
           g2, b2, g3, b3):
    xs = x[0]
    xbf = xs.astype(jnp.bfloat16)

    logits = (x @ Wr.T + br)[0]

    pos1, pos2, w1c, w2c, eb, meta, aux = pl.pallas_call(
        _router_body,
        grid=(1,),
        in_specs=[pl.BlockSpec((S, E), lambda i: (0, 0))],
        out_specs=[
            pl.BlockSpec((S, 1), lambda i: (0, 0)),
            pl.BlockSpec((S, 1), lambda i: (0, 0)),
            pl.BlockSpec((S, 1), lambda i: (0, 0)),
            pl.BlockSpec((S, 1), lambda i: (0, 0)),
            pl.BlockSpec((1, NBPAD), lambda i: (0, 0)),
            pl.BlockSpec((2, 16), lambda i: (0, 0)),
            pl.BlockSpec((1, 1), lambda i: (0, 0)),
        ],
        out_shape=[
            jax.ShapeDtypeStruct((S, 1), jnp.int32),
            jax.ShapeDtypeStruct((S, 1), jnp.int32),
            jax.ShapeDtypeStruct((S, 1), jnp.float32),
            jax.ShapeDtypeStruct((S, 1), jnp.float32),
            jax.ShapeDtypeStruct((1, NBPAD), jnp.int32),
            jax.ShapeDtypeStruct((2, 16), jnp.int32),
            jax.ShapeDtypeStruct((1, 1), jnp.float32),
        ],
    )(logits)

    p1 = pos1.reshape(S)
    p2 = pos2.reshape(S)

    tokpos, wpos = _sc_scatter(p1, p2, w1c.reshape(S), w2c.reshape(S), meta)
    x_i32 = jax.lax.bitcast_convert_type(
        xbf.reshape(S, D // 2, 2), jnp.int32)
    xg_i32 = _sc_gather(tokpos, x_i32)
    xg = jax.lax.bitcast_convert_type(
        xg_i32, jnp.bfloat16).reshape(NPOS, D)
    wpos = wpos[:NPOS]

    kv = pl.pallas_call(
        _kv_body,
        grid=(E,),
        in_specs=[
            pl.BlockSpec((S, D), lambda e: (0, 0)),
            pl.BlockSpec((1, 2 * D, D), lambda e: (e, 0, 0)),
            pl.BlockSpec((1, 8, 2 * D), lambda e: (e, 0, 0)),
        ],
        out_specs=pl.BlockSpec((1, S, 2 * D), lambda e: (e, 0, 0)),
        out_shape=jax.ShapeDtypeStruct((E, S, 2 * D), jnp.bfloat16),
    )(xbf, Wqkv[:, D:, :].astype(jnp.bfloat16), _rep8(bqkv[:, D:]))

    zg = pl.pallas_call(
        _gexpert_body,
        grid_spec=pltpu.PrefetchScalarGridSpec(
            num_scalar_prefetch=1,
            grid=(NB,),
            in_specs=[
                pl.BlockSpec((RB, D), lambda b, eb: (b, 0)),
                pl.BlockSpec((1, S, 2 * D), lambda b, eb: (eb[b], 0, 0)),
                pl.BlockSpec((RB, 1), lambda b, eb: (b, 0)),
                pl.BlockSpec((1, D, D), lambda b, eb: (eb[b], 0, 0)),
                pl.BlockSpec((1, 8, 3 * D), lambda b, eb: (eb[b], 0, 0)),
                pl.BlockSpec((1, D, D), lambda b, eb: (eb[b], 0, 0)),
                pl.BlockSpec((1, 8, D), lambda b, eb: (eb[b], 0, 0)),
                pl.BlockSpec((1, 8, D), lambda b, eb: (eb[b], 0, 0)),
                pl.BlockSpec((1, 8, D), lambda b, eb: (eb[b], 0, 0)),
                pl.BlockSpec((1, F, D), lambda b, eb: (eb[b], 0, 0)),
                pl.BlockSpec((1, 8, F), lambda b, eb: (eb[b], 0, 0)),
                pl.BlockSpec((1, D, F), lambda b, eb: (eb[b], 0, 0)),
                pl.BlockSpec((1, 8, D), lambda b, eb: (eb[b], 0, 0)),
                pl.BlockSpec((1, 8, D), lambda b, eb: (eb[b], 0, 0)),
                pl.BlockSpec((1, 8, D), lambda b, eb: (eb[b], 0, 0)),
                pl.BlockSpec((1, 8, D), lambda b, eb: (eb[b], 0, 0)),
                pl.BlockSpec((1, 8, D), lambda b, eb: (eb[b], 0, 0)),
            ],
            out_specs=pl.BlockSpec((RB, D), lambda b, eb: (b, 0)),
        ),
        out_shape=jax.ShapeDtypeStruct((NPOS, D), jnp.float32),
    )(eb[0], xg, kv, wpos.reshape(NPOS, 1),
      Wqkv[:, :D, :].astype(jnp.bfloat16), _rep8(bqkv),
      Wo.astype(jnp.bfloat16), _rep8(bo), _rep8(g1), _rep8(b1),
      Wf1.astype(jnp.bfloat16), _rep8(bf1), Wf2.astype(jnp.bfloat16),
      _rep8(bf2), _rep8(g2), _rep8(b2), _rep8(g3), _rep8(b3))

    out2d = _sc_combine(zg, p1, p2)

    return out2d[None], aux[0, 0]


# speedup vs baseline: 1.2948x; 1.2652x over previous
import functools
import math

import jax
import jax.numpy as jnp
from jax import lax
from jax.experimental import pallas as pl
from jax.experimental.pallas import tpu as pltpu
from jax.experimental.pallas import tpu_sc as plsc

E = 8
K = 2
D = 768
H = 12
DH = D // H
S = 2048
F = 4 * D

RB = 128
NPOS = 5120
NB = NPOS // RB
NBPAD = 64


def _router_body(lg_ref, pos1_ref, pos2_ref, w1_ref, w2_ref, eb_ref,
                 meta_ref, aux_ref):
    logits = lg_ref[...]
    m = jnp.max(logits, axis=1, keepdims=True)
    ex = jnp.exp(logits - m)
    probs = ex / jnp.sum(ex, axis=1, keepdims=True)

    lane = jax.lax.broadcasted_iota(jnp.int32, (S, E), 1)
    i1 = jnp.min(jnp.where(logits == m, lane, E), axis=1, keepdims=True)
    sel1 = lane == i1
    logits2 = jnp.where(sel1, jnp.float32(-jnp.inf), logits)
    m2 = jnp.max(logits2, axis=1, keepdims=True)
    i2 = jnp.min(jnp.where(logits2 == m2, lane, E), axis=1, keepdims=True)
    sel2 = lane == i2
    v1 = jnp.max(probs, axis=1, keepdims=True)
    v2 = jnp.max(jnp.where(sel1, -1.0, probs), axis=1, keepdims=True)
    wsum = jnp.maximum(v1 + v2, 1e-9)
    w1_ref[...] = v1 / wsum
    w2_ref[...] = v2 / wsum

    A = (sel1 | sel2).astype(jnp.float32)
    c = A
    sh = 1
    while sh < S:
        c = c + jnp.concatenate(
            [jnp.zeros((sh, E), jnp.float32), c[:S - sh]], axis=0)
        sh *= 2
    rank = c - A
    n = c[S - 1:S, :]
    mpad = jnp.floor((n + (RB - 1.0)) * (1.0 / RB)) * float(RB)
    mc = mpad
    sh = 1
    while sh < E:
        mc = mc + jnp.concatenate(
            [jnp.zeros((1, sh), jnp.float32), mc[:, :E - sh]], axis=1)
        sh *= 2
    off = mc - mpad
    total = mc[:, E - 1:E]

    pos = off + rank
    p1 = jnp.sum(jnp.where(sel1, pos, 0.0), axis=1, keepdims=True)
    p2 = jnp.sum(jnp.where(sel2, pos, 0.0), axis=1, keepdims=True)
    pos1_ref[...] = p1.astype(jnp.int32)
    pos2_ref[...] = p2.astype(jnp.int32)

    bi = jax.lax.broadcasted_iota(jnp.int32, (1, NBPAD), 1).astype(jnp.float32) * float(RB)
    end = off + mpad
    acc = jnp.zeros((1, NBPAD), jnp.float32)
    for e in range(E):
        acc = acc + (bi >= end[:, e:e + 1]).astype(jnp.float32)
    ebv = jnp.minimum(acc, float(E - 1))
    lanei = jax.lax.broadcasted_iota(jnp.int32, (1, NBPAD), 1)
    ebv = jnp.where(lanei == NBPAD - 1, total * (1.0 / RB), ebv)
    eb_ref[...] = ebv.astype(jnp.int32)

    r0 = jnp.concatenate([off, total, jnp.zeros((1, 7), jnp.float32)], axis=1)
    r1 = jnp.concatenate([n, jnp.zeros((1, 8), jnp.float32)], axis=1)
    meta_ref[...] = jnp.concatenate([r0, r1], axis=0).astype(jnp.int32)

    f = jnp.sum(sel1.astype(jnp.float32), axis=0, keepdims=True) / S
    p_mean = jnp.sum(probs, axis=0, keepdims=True) / S
    aux_ref[...] = float(E) * jnp.sum(f * p_mean, axis=1, keepdims=True)


def _ln(x, g, b, eps=1e-5):
    mu = jnp.mean(x, axis=-1, keepdims=True)
    xc = x - mu
    var = jnp.mean(xc * xc, axis=-1, keepdims=True)
    return xc * jax.lax.rsqrt(var + eps) * g + b


def _kv_body(xkv_ref, wkv_ref, bkv_ref, kv_ref):
    cdims = (((1,), (1,)), ((), ()))
    kv = jax.lax.dot_general(xkv_ref[...], wkv_ref[0], cdims,
                             preferred_element_type=jnp.float32)
    kv_ref[0] = (kv + bkv_ref[0, 0, :][None, :]).astype(jnp.bfloat16)


def _gexpert_body(eb_ref, tok_ref, xkv_ref, kv_ref, w_ref, wqkv_ref, bqkv_ref,
                  wo_ref, bo_ref, g1_ref, b1_ref, wf1_ref, bf1_ref, wf2_ref,
                  bf2_ref, g2_ref, b2_ref, g3_ref, b3_ref, out_ref):
    b = pl.program_id(0)

    @pl.when(b >= eb_ref[NBPAD - 1])
    def _():
        out_ref[...] = jnp.zeros((RB, D), jnp.bfloat16)

    @pl.when(b < eb_ref[NBPAD - 1])
    def _():
        cdims = (((1,), (1,)), ((), ()))
        wqkv = wqkv_ref[0]
        brow = bqkv_ref[0, 0, :][None, :]

        tok_iota = jax.lax.broadcasted_iota(jnp.int32, (RB, S), 1)
        sel = (tok_iota == tok_ref[...]).astype(jnp.bfloat16)
        xq = jnp.dot(sel, xkv_ref[...],
                     preferred_element_type=jnp.float32).astype(jnp.bfloat16)
        q = jax.lax.dot_general(xq, wqkv, cdims,
                                preferred_element_type=jnp.float32)
        qb = (q + brow[:, :D]).astype(jnp.bfloat16)

        heads = []
        for h in range(H):
            cols = slice(h * DH, (h + 1) * DH)
            colsv = slice(D + h * DH, D + (h + 1) * DH)
            s = jax.lax.dot_general(qb[:, cols], kv_ref[0][:, cols], cdims,
                                    preferred_element_type=jnp.float32)
            s = s * (1.0 / math.sqrt(DH))
            mx = jnp.max(s, axis=1, keepdims=True)
            p = jnp.exp(s - mx)
            ps = jnp.sum(p, axis=1, keepdims=True)
            o = jax.lax.dot_general(p.astype(jnp.bfloat16), kv_ref[0][:, colsv],
                                    (((1,), (0,)), ((), ())),
                                    preferred_element_type=jnp.float32)
            heads.append((o / ps).astype(jnp.bfloat16))
        attn = jnp.concatenate(heads, axis=1)

        xb = xq.astype(jnp.float32)
        a = jax.lax.dot_general(attn, wo_ref[0], cdims,
                                preferred_element_type=jnp.float32)
        a = a + bo_ref[0, 0, :][None, :]
        h1 = _ln(xb + a, g1_ref[0, 0, :][None, :], b1_ref[0, 0, :][None, :])
        ff = jax.lax.dot_general(h1.astype(jnp.bfloat16), wf1_ref[0], cdims,
                                 preferred_element_type=jnp.float32)
        ff = ff + bf1_ref[0, 0, :][None, :]
        ff = 0.5 * ff * (1.0 + jax.lax.erf(ff * (1.0 / math.sqrt(2.0))))
        y = jax.lax.dot_general(ff.astype(jnp.bfloat16), wf2_ref[0], cdims,
                                preferred_element_type=jnp.float32)
        y = y + bf2_ref[0, 0, :][None, :]
        t = _ln(h1 + y, g2_ref[0, 0, :][None, :], b2_ref[0, 0, :][None, :])
        z = _ln(t + xb, g3_ref[0, 0, :][None, :], b3_ref[0, 0, :][None, :])
        out_ref[...] = (z * w_ref[...]).astype(jnp.bfloat16)


def _combine_body(p1_ref, p2_ref, zg_ref, out_ref):
    pos_iota = jax.lax.broadcasted_iota(jnp.int32, (RB, NPOS), 1)
    sel = (jnp.logical_or(pos_iota == p1_ref[...], pos_iota == p2_ref[...])
           ).astype(jnp.bfloat16)
    out_ref[...] = jnp.dot(sel, zg_ref[...],
                           preferred_element_type=jnp.float32)


def _rep8(b):
    return jnp.broadcast_to(b[:, None, :], (b.shape[0], 8, b.shape[1]))


NC = 2
NS = 16
NW = NC * NS
TOKT = S // NW
POST = NPOS // NW
NTRASH = 8
_SC_MESH = plsc.VectorSubcoreMesh(core_axis_name="c", subcore_axis_name="s")


def _wid():
    return lax.axis_index("s") * NC + lax.axis_index("c")


@functools.partial(
    pl.kernel, mesh=_SC_MESH,
    compiler_params=pltpu.CompilerParams(needs_layout_passes=False),
    out_type=[
        jax.ShapeDtypeStruct((NPOS + NTRASH,), jnp.int32),
        jax.ShapeDtypeStruct((NPOS + NTRASH,), jnp.float32),
    ],
    scratch_types=[
        pltpu.VMEM((TOKT,), jnp.int32),
        pltpu.VMEM((TOKT,), jnp.int32),
        pltpu.VMEM((TOKT,), jnp.int32),
        pltpu.VMEM((TOKT,), jnp.float32),
        pltpu.VMEM((TOKT,), jnp.float32),
        pltpu.VMEM((RB,), jnp.int32),
        pltpu.VMEM((RB,), jnp.int32),
        pltpu.VMEM((RB,), jnp.float32),
        pltpu.VMEM((16,), jnp.int32),
        pltpu.VMEM((16,), jnp.int32),
    ],
)
def _sc_scatter(pos1_hbm, pos2_hbm, w1_hbm, w2_hbm, meta_hbm,
                tokpos_hbm, wpos_hbm,
                idx1_v, idx2_v, tok_v, wv1_v, wv2_v, fidx_v, fsrc_v,
                fsrcw_v, m0_v, m1_v):
    wid = _wid()
    base = wid * TOKT
    pltpu.sync_copy(pos1_hbm.at[pl.ds(base, TOKT)], idx1_v)
    pltpu.sync_copy(pos2_hbm.at[pl.ds(base, TOKT)], idx2_v)
    pltpu.sync_copy(w1_hbm.at[pl.ds(base, TOKT)], wv1_v)
    pltpu.sync_copy(w2_hbm.at[pl.ds(base, TOKT)], wv2_v)
    for c in range(TOKT // 16):
        tok_v[pl.ds(c * 16, 16)] = (
            jnp.arange(16, dtype=jnp.int32) + (base + c * 16))
    pltpu.sync_copy(tok_v, tokpos_hbm.at[idx1_v])
    pltpu.sync_copy(tok_v, tokpos_hbm.at[idx2_v])
    pltpu.sync_copy(wv1_v, wpos_hbm.at[idx1_v])
    pltpu.sync_copy(wv2_v, wpos_hbm.at[idx2_v])

    trash = NPOS + lax.rem(wid, NTRASH)

    @pl.when(wid < E)
    def _():
        pltpu.sync_copy(meta_hbm.at[0], m0_v)
        pltpu.sync_copy(meta_hbm.at[1], m1_v)
        lanes = jnp.arange(16, dtype=jnp.int32)
        offv = jnp.max(jnp.where(lanes == wid, m0_v[...], 0), axis=0)
        nv = jnp.max(jnp.where(lanes == wid, m1_v[...], 0), axis=0)
        mv = ((nv + (RB - 1)) // RB) * RB
        for c in range(RB // 16):
            j = jnp.arange(16, dtype=jnp.int32) + c * 16
            valid = (nv + j) < mv
            fidx_v[pl.ds(c * 16, 16)] = jnp.where(
                valid, offv + nv + j, trash)
            fsrc_v[pl.ds(c * 16, 16)] = jnp.zeros((16,), jnp.int32)
            fsrcw_v[pl.ds(c * 16, 16)] = jnp.zeros((16,), jnp.float32)
        pltpu.sync_copy(fsrc_v, tokpos_hbm.at[fidx_v])
        pltpu.sync_copy(fsrcw_v, wpos_hbm.at[fidx_v])

    @pl.when(jnp.logical_and(wid >= E, wid < 2 * E))
    def _():
        pltpu.sync_copy(meta_hbm.at[0], m0_v)
        lanes = jnp.arange(16, dtype=jnp.int32)
        totv = jnp.max(jnp.where(lanes == E, m0_v[...], 0), axis=0)
        for c in range(RB // 16):
            p = (jnp.arange(16, dtype=jnp.int32)
                 + (2 * S + (wid - E) * RB + c * 16))
            fidx_v[pl.ds(c * 16, 16)] = jnp.where(p >= totv, p, trash)
            fsrc_v[pl.ds(c * 16, 16)] = jnp.zeros((16,), jnp.int32)
            fsrcw_v[pl.ds(c * 16, 16)] = jnp.zeros((16,), jnp.float32)
        pltpu.sync_copy(fsrc_v, tokpos_hbm.at[fidx_v])
        pltpu.sync_copy(fsrcw_v, wpos_hbm.at[fidx_v])


def kernel(x, Wr, br, Wqkv, bqkv, Wo, bo, g1, b1, Wf1, bf1, Wf2, bf2,
           g2, b2, g3, b3):
    xs = x[0]
    xbf = xs.astype(jnp.bfloat16)

    logits = (x @ Wr.T + br)[0]

    pos1, pos2, w1c, w2c, eb, meta, aux = pl.pallas_call(
        _router_body,
        grid=(1,),
        in_specs=[pl.BlockSpec((S, E), lambda i: (0, 0))],
        out_specs=[
            pl.BlockSpec((S, 1), lambda i: (0, 0)),
            pl.BlockSpec((S, 1), lambda i: (0, 0)),
            pl.BlockSpec((S, 1), lambda i: (0, 0)),
            pl.BlockSpec((S, 1), lambda i: (0, 0)),
            pl.BlockSpec((1, NBPAD), lambda i: (0, 0)),
            pl.BlockSpec((2, 16), lambda i: (0, 0)),
            pl.BlockSpec((1, 1), lambda i: (0, 0)),
        ],
        out_shape=[
            jax.ShapeDtypeStruct((S, 1), jnp.int32),
            jax.ShapeDtypeStruct((S, 1), jnp.int32),
            jax.ShapeDtypeStruct((S, 1), jnp.float32),
            jax.ShapeDtypeStruct((S, 1), jnp.float32),
            jax.ShapeDtypeStruct((1, NBPAD), jnp.int32),
            jax.ShapeDtypeStruct((2, 16), jnp.int32),
            jax.ShapeDtypeStruct((1, 1), jnp.float32),
        ],
    )(logits)

    p1 = pos1.reshape(S)
    p2 = pos2.reshape(S)

    tokpos, wpos = _sc_scatter(p1, p2, w1c.reshape(S), w2c.reshape(S), meta)
    tok2d = tokpos[:NPOS].reshape(NPOS, 1)
    wpos = wpos[:NPOS]

    kv = pl.pallas_call(
        _kv_body,
        grid=(E,),
        in_specs=[
            pl.BlockSpec((S, D), lambda e: (0, 0)),
            pl.BlockSpec((1, 2 * D, D), lambda e: (e, 0, 0)),
            pl.BlockSpec((1, 8, 2 * D), lambda e: (e, 0, 0)),
        ],
        out_specs=pl.BlockSpec((1, S, 2 * D), lambda e: (e, 0, 0)),
        out_shape=jax.ShapeDtypeStruct((E, S, 2 * D), jnp.bfloat16),
    )(xbf, Wqkv[:, D:, :].astype(jnp.bfloat16), _rep8(bqkv[:, D:]))

    zg = pl.pallas_call(
        _gexpert_body,
        grid_spec=pltpu.PrefetchScalarGridSpec(
            num_scalar_prefetch=1,
            grid=(NB,),
            in_specs=[
                pl.BlockSpec((RB, 1), lambda b, eb: (b, 0)),
                pl.BlockSpec((S, D), lambda b, eb: (0, 0)),
                pl.BlockSpec((1, S, 2 * D), lambda b, eb: (eb[b], 0, 0)),
                pl.BlockSpec((RB, 1), lambda b, eb: (b, 0)),
                pl.BlockSpec((1, D, D), lambda b, eb: (eb[b], 0, 0)),
                pl.BlockSpec((1, 8, 3 * D), lambda b, eb: (eb[b], 0, 0)),
                pl.BlockSpec((1, D, D), lambda b, eb: (eb[b], 0, 0)),
                pl.BlockSpec((1, 8, D), lambda b, eb: (eb[b], 0, 0)),
                pl.BlockSpec((1, 8, D), lambda b, eb: (eb[b], 0, 0)),
                pl.BlockSpec((1, 8, D), lambda b, eb: (eb[b], 0, 0)),
                pl.BlockSpec((1, F, D), lambda b, eb: (eb[b], 0, 0)),
                pl.BlockSpec((1, 8, F), lambda b, eb: (eb[b], 0, 0)),
                pl.BlockSpec((1, D, F), lambda b, eb: (eb[b], 0, 0)),
                pl.BlockSpec((1, 8, D), lambda b, eb: (eb[b], 0, 0)),
                pl.BlockSpec((1, 8, D), lambda b, eb: (eb[b], 0, 0)),
                pl.BlockSpec((1, 8, D), lambda b, eb: (eb[b], 0, 0)),
                pl.BlockSpec((1, 8, D), lambda b, eb: (eb[b], 0, 0)),
                pl.BlockSpec((1, 8, D), lambda b, eb: (eb[b], 0, 0)),
            ],
            out_specs=pl.BlockSpec((RB, D), lambda b, eb: (b, 0)),
        ),
        out_shape=jax.ShapeDtypeStruct((NPOS, D), jnp.bfloat16),
    )(eb[0], tok2d, xbf, kv, wpos.reshape(NPOS, 1),
      Wqkv[:, :D, :].astype(jnp.bfloat16), _rep8(bqkv),
      Wo.astype(jnp.bfloat16), _rep8(bo), _rep8(g1), _rep8(b1),
      Wf1.astype(jnp.bfloat16), _rep8(bf1), Wf2.astype(jnp.bfloat16),
      _rep8(bf2), _rep8(g2), _rep8(b2), _rep8(g3), _rep8(b3))

    out2d = pl.pallas_call(
        _combine_body,
        grid=(S // RB,),
        in_specs=[
            pl.BlockSpec((RB, 1), lambda t: (t, 0)),
            pl.BlockSpec((RB, 1), lambda t: (t, 0)),
            pl.BlockSpec((NPOS, D), lambda t: (0, 0)),
        ],
        out_specs=pl.BlockSpec((RB, D), lambda t: (t, 0)),
        out_shape=jax.ShapeDtypeStruct((S, D), jnp.float32),
    )(pos1, pos2, zg)

    return out2d[None], aux[0, 0]
